# Initial kernel scaffold; baseline (speedup 1.0000x reference)
import jax
import jax.numpy as jnp
from jax.experimental import pallas as pl


def kernel(z, codebook, codebook_t):
    raise NotImplementedError("write your pallas kernel here")


# speedup vs baseline: 2.1133x; 2.1133x over previous
import functools

import jax
import jax.numpy as jnp
from jax.experimental import pallas as pl
from jax.experimental.pallas import tpu as pltpu

DEPTH = 4
B_TOK = 16384
CODE_DIM = 256
N_CODES = 1024
BLK = 256


def _vq_body(z_ref, cb_ref, ct_ref, cn_ref, zq_ref, m0, m1, m2, m3):
    maps = (m0, m1, m2, m3)
    r = z_ref[...]
    zq = jnp.zeros_like(r)
    cn = cn_ref[...]
    for i in range(DEPTH):
        rn = jnp.sum(r * r, axis=1, keepdims=True)
        g = jax.lax.dot_general(
            r, ct_ref[...], (((1,), (1,)), ((), ())),
            preferred_element_type=jnp.float32)
        dist = (rn + cn) - 2.0 * g
        maps[i][...] = dist
        idx = jnp.argmin(dist, axis=1)
        oh = (jax.lax.broadcasted_iota(jnp.int32, (BLK, N_CODES), 1)
              == idx[:, None]).astype(jnp.float32)
        delta = jax.lax.dot_general(
            oh, cb_ref[...], (((1,), (0,)), ((), ())),
            preferred_element_type=jnp.float32)
        zq = zq + delta
        r = r - delta
    zq_ref[...] = zq


@jax.jit
def kernel(z, codebook, codebook_t):
    cn = jnp.sum(codebook_t ** 2, axis=1)[None, :]
    grid = (B_TOK // BLK,)
    map_spec = pl.BlockSpec((BLK, N_CODES), lambda b: (b, 0))
    out = pl.pallas_call(
        _vq_body,
        grid=grid,
        in_specs=[
            pl.BlockSpec((BLK, CODE_DIM), lambda b: (b, 0)),
            pl.BlockSpec((N_CODES, CODE_DIM), lambda b: (0, 0)),
            pl.BlockSpec((N_CODES, CODE_DIM), lambda b: (0, 0)),
            pl.BlockSpec((1, N_CODES), lambda b: (0, 0)),
        ],
        out_specs=[
            pl.BlockSpec((BLK, CODE_DIM), lambda b: (b, 0)),
            map_spec, map_spec, map_spec, map_spec,
        ],
        out_shape=[
            jax.ShapeDtypeStruct((B_TOK, CODE_DIM), jnp.float32),
        ] + [jax.ShapeDtypeStruct((B_TOK, N_CODES), jnp.float32)] * DEPTH,
        compiler_params=pltpu.CompilerParams(
            dimension_semantics=("parallel",)),
    )(z, codebook, codebook_t, cn)
    return tuple(out)
